# baseline (speedup 1.0000x reference)
import functools

import jax
import jax.numpy as jnp
from jax import lax
from jax.experimental import pallas as pl
from jax.experimental.pallas import tpu as pltpu
from jax.experimental.pallas import tpu_sc as plsc

NC = 2
NS = 16
NW = NC * NS
CH = 125
GRP = 2
NH = 2
ZB = 64


def _pad_rows(n):
    rows_per_tile = -(-(-(-n // NS)) // ZB) * ZB
    return rows_per_tile, rows_per_tile * NS


@functools.cache
def _make_agg(n, n_pad, d, e):
    chunks_total = e // CH
    chunks_per_tile = chunks_total // NW
    rows_per_tile = n_pad // NS
    reps = rows_per_tile // ZB
    chunks_per_half = chunks_per_tile // NH
    niter = chunks_per_half // (2 * GRP)

    mesh = plsc.VectorSubcoreMesh(
        core_axis_name="c", subcore_axis_name="s", num_cores=NC, num_subcores=NS
    )

    @functools.partial(
        pl.kernel,
        out_type=jax.ShapeDtypeStruct((NC, n_pad, d), jnp.float32),
        mesh=mesh,
        scratch_types=[
            pltpu.VMEM((chunks_per_half, CH), jnp.int32),
            pltpu.VMEM((chunks_per_half, CH), jnp.int32),
            pltpu.VMEM((2, GRP, CH, d), jnp.float32),
            pltpu.VMEM((ZB, d), jnp.float32),
            pltpu.VMEM_SHARED((n, d), jnp.float32),
            pltpu.VMEM_SHARED((n_pad, d), jnp.float32),
            pltpu.SemaphoreType.DMA,
            pltpu.SemaphoreType.DMA,
            pltpu.SemaphoreType.DMA,
            pltpu.SemaphoreType.DMA,
            pltpu.SemaphoreType.DMA,
        ],
        compiler_params=pltpu.CompilerParams(use_tc_tiling_on_sc=False),
    )
    def agg(xb_hbm, src_hbm, dst_hbm, out_hbm, src_v, dst_v, rows_v, zb_v,
            stage_sh, acc_sh, gsem0, gsem1, ssem0, ssem1, isem):
        c = lax.axis_index("c")
        s = lax.axis_index("s")
        wid = c * NS + s
        my_rows = pl.ds(s * rows_per_tile, rows_per_tile)

        pltpu.sync_copy(xb_hbm.at[pl.ds(s * (n // NS), n // NS)],
                        stage_sh.at[pl.ds(s * (n // NS), n // NS)])

        def zrow(i, carry):
            for jj in range(d // 16):
                zb_v[i, pl.ds(jj * 16, 16)] = jnp.zeros((16,), jnp.float32)
            return carry

        lax.fori_loop(0, ZB, zrow, 0)
        for r in range(reps):
            pltpu.sync_copy(zb_v, acc_sh.at[pl.ds(s * rows_per_tile + r * ZB, ZB)])
        plsc.subcore_barrier()

        def fire_gathers(half, grp, sem):
            return [
                pltpu.async_copy(
                    stage_sh.at[src_v.at[grp * GRP + b]], rows_v.at[half, b],
                    sem)
                for b in range(GRP)
            ]

        def fire_scatters(half, grp, sem):
            return [
                pltpu.async_copy(
                    rows_v.at[half, b], acc_sh.at[dst_v.at[grp * GRP + b]],
                    sem, add=True)
                for b in range(GRP)
            ]

        def body(i, carry):
            ga = 2 * i
            gb = 2 * i + 1
            gA = fire_gathers(0, ga, gsem0)
            for cp in gA:
                cp.wait()
            sA = fire_scatters(0, ga, ssem0)
            gB = fire_gathers(1, gb, gsem1)
            for cp in gB:
                cp.wait()
            for cp in sA:
                cp.wait()
            sB = fire_scatters(1, gb, ssem1)
            for cp in sB:
                cp.wait()
            return carry

        for hf in range(NH):
            pltpu.sync_copy(
                src_hbm.at[pl.ds(wid * chunks_per_tile + hf * chunks_per_half,
                                 chunks_per_half)], src_v)
            pltpu.sync_copy(
                dst_hbm.at[pl.ds(wid * chunks_per_tile + hf * chunks_per_half,
                                 chunks_per_half)], dst_v)
            lax.fori_loop(0, niter, body, 0)
        plsc.subcore_barrier()

        pltpu.sync_copy(acc_sh.at[my_rows], out_hbm.at[c].at[my_rows])

    return agg


def _gin_mlp(n, sfull, w1_ref, b1_ref, g_ref, be_ref, w2_ref, b2_ref,
             out_ref):
    t = jnp.dot(sfull, w1_ref[...], preferred_element_type=jnp.float32)
    t = t + b1_ref[...]
    mu = jnp.sum(t, axis=0, keepdims=True) * (1.0 / n)
    ctr = t - mu
    var = jnp.sum(ctr * ctr, axis=0, keepdims=True) * (1.0 / n)
    h = ctr * lax.rsqrt(var + 1e-5) * g_ref[...] + be_ref[...]
    h = jnp.maximum(h, 0.0)
    h = jnp.dot(h, w2_ref[...], preferred_element_type=jnp.float32) + b2_ref[...]
    out_ref[...] = jnp.maximum(h, 0.0)


def _gin_dense1_body(n, x_ref, pa_ref, pb_ref, w1_ref, b1_ref, g_ref,
                     be_ref, w2_ref, b2_ref, out_ref):
    agg = jnp.concatenate(
        [pa_ref[0][:n] + pa_ref[1][:n], pb_ref[0][:n] + pb_ref[1][:n]], axis=1
    )
    sfull = x_ref[...] + agg
    _gin_mlp(n, sfull, w1_ref, b1_ref, g_ref, be_ref, w2_ref, b2_ref, out_ref)


def _gin_dense_body(n, x_ref, a_ref, w1_ref, b1_ref, g_ref, be_ref,
                    w2_ref, b2_ref, out_ref):
    sfull = x_ref[...] + a_ref[0][:n] + a_ref[1][:n]
    _gin_mlp(n, sfull, w1_ref, b1_ref, g_ref, be_ref, w2_ref, b2_ref, out_ref)


@functools.cache
def _make_gin_dense1(n, h):
    return pl.pallas_call(
        functools.partial(_gin_dense1_body, n),
        out_shape=jax.ShapeDtypeStruct((n, h), jnp.float32),
    )


@functools.cache
def _make_gin_dense(n, h):
    return pl.pallas_call(
        functools.partial(_gin_dense_body, n),
        out_shape=jax.ShapeDtypeStruct((n, h), jnp.float32),
    )


def _dense3_pool_head_body(n, g, x_ref, a_ref, w1_ref, b1_ref, g_ref, be_ref,
                           w2_ref, b2_ref, h1_ref, h2_ref, batch_ref,
                           fc1w_ref, fc1b_ref, fc2w_ref, fc2b_ref, out_ref):
    sfull = x_ref[...] + a_ref[0][:n] + a_ref[1][:n]
    t = jnp.dot(sfull, w1_ref[...], preferred_element_type=jnp.float32)
    t = t + b1_ref[...]
    mu = jnp.sum(t, axis=0, keepdims=True) * (1.0 / n)
    ctr = t - mu
    var = jnp.sum(ctr * ctr, axis=0, keepdims=True) * (1.0 / n)
    hh = ctr * lax.rsqrt(var + 1e-5) * g_ref[...] + be_ref[...]
    hh = jnp.maximum(hh, 0.0)
    hh = jnp.dot(hh, w2_ref[...], preferred_element_type=jnp.float32)
    h3 = jnp.maximum(hh + b2_ref[...], 0.0)

    b = batch_ref[...]
    gids = lax.broadcasted_iota(jnp.int32, (g, n), 0)
    sel = jnp.where(gids == b, 1.0, 0.0)
    p1 = jnp.dot(sel, h1_ref[...], preferred_element_type=jnp.float32)
    p2 = jnp.dot(sel, h2_ref[...], preferred_element_type=jnp.float32)
    p3 = jnp.dot(sel, h3, preferred_element_type=jnp.float32)
    cat = jnp.concatenate([p1, p2, p3], axis=1)
    y = jnp.dot(cat, fc1w_ref[...], preferred_element_type=jnp.float32)
    y = jnp.maximum(y + fc1b_ref[...], 0.0)
    y = jnp.dot(y, fc2w_ref[...], preferred_element_type=jnp.float32)
    y = y + fc2b_ref[...]
    m = jnp.max(y, axis=1, keepdims=True)
    ex = jnp.exp(y - m)
    out_ref[...] = (y - m) - jnp.log(jnp.sum(ex, axis=1, keepdims=True))


@functools.cache
def _make_dense3_pool_head(n, g, out):
    return pl.pallas_call(
        functools.partial(_dense3_pool_head_body, n, g),
        out_shape=jax.ShapeDtypeStruct((g, out), jnp.float32),
    )


def kernel(x, edge_index, batch, c1_W1, c1_b1, c1_g, c1_be, c1_W2, c1_b2,
           c2_W1, c2_b1, c2_g, c2_be, c2_W2, c2_b2, c3_W1, c3_b1, c3_g,
           c3_be, c3_W2, c3_b2, fc1_W, fc1_b, fc2_W, fc2_b):
    n, d = x.shape
    e = edge_index.shape[1]
    h = c1_W1.shape[1]
    g = 64
    out = fc2_W.shape[1]
    _, n_pad = _pad_rows(n)

    src2d = edge_index[0].reshape(e // CH, CH)
    dst2d = edge_index[1].reshape(e // CH, CH)

    xa = x[:, :h]
    xb = x[:, h:]

    agg = _make_agg(n, n_pad, h, e)
    dense1 = _make_gin_dense1(n, h)
    dense23 = _make_gin_dense(n, h)
    dense3_pool = _make_dense3_pool_head(n, g, out)

    pa = agg(xa, src2d, dst2d)
    pb = agg(xb, src2d, dst2d)
    h1 = dense1(x, pa, pb, c1_W1, c1_b1.reshape(1, h), c1_g.reshape(1, h),
                c1_be.reshape(1, h), c1_W2, c1_b2.reshape(1, h))

    parts2 = agg(h1, src2d, dst2d)
    h2 = dense23(h1, parts2, c2_W1, c2_b1.reshape(1, h), c2_g.reshape(1, h),
                 c2_be.reshape(1, h), c2_W2, c2_b2.reshape(1, h))

    parts3 = agg(h2, src2d, dst2d)
    return dense3_pool(h2, parts3, c3_W1, c3_b1.reshape(1, h),
                       c3_g.reshape(1, h), c3_be.reshape(1, h), c3_W2,
                       c3_b2.reshape(1, h), h1, h2, batch.reshape(1, n),
                       fc1_W, fc1_b.reshape(1, 3 * h), fc2_W,
                       fc2_b.reshape(1, out))


# speedup vs baseline: 1.2410x; 1.2410x over previous
import functools

import jax
import jax.numpy as jnp
from jax import lax
from jax.experimental import pallas as pl
from jax.experimental.pallas import tpu as pltpu
from jax.experimental.pallas import tpu_sc as plsc

NC = 2
NS = 16
NW = NC * NS
CH = 125
GRP = 4
ZB = 64


def _pad_rows(n):
    rows_per_tile = -(-(-(-n // NS)) // ZB) * ZB
    return rows_per_tile, rows_per_tile * NS


@functools.cache
def _make_agg(n, n_pad, d, e):
    chunks_total = e // CH
    chunks_per_tile = chunks_total // NW
    rows_per_tile = n_pad // NS
    reps = rows_per_tile // ZB
    niter = chunks_per_tile // (2 * GRP)

    mesh = plsc.VectorSubcoreMesh(
        core_axis_name="c", subcore_axis_name="s", num_cores=NC, num_subcores=NS
    )

    @functools.partial(
        pl.kernel,
        out_type=jax.ShapeDtypeStruct((NC, n_pad, d), jnp.float32),
        mesh=mesh,
        scratch_types=[
            pltpu.VMEM((chunks_per_tile, CH), jnp.int32),
            pltpu.VMEM((chunks_per_tile, CH), jnp.int32),
            pltpu.VMEM((2, GRP, CH, d), jnp.float32),
            pltpu.VMEM((ZB, d), jnp.float32),
            pltpu.VMEM_SHARED((n_pad, d), jnp.float32),
            pltpu.SemaphoreType.DMA,
            pltpu.SemaphoreType.DMA,
            pltpu.SemaphoreType.DMA,
            pltpu.SemaphoreType.DMA,
            pltpu.SemaphoreType.DMA,
        ],
        compiler_params=pltpu.CompilerParams(use_tc_tiling_on_sc=False),
    )
    def agg(xb_hbm, src_hbm, dst_hbm, out_hbm, src_v, dst_v, rows_v, zb_v,
            acc_sh, gsem0, gsem1, ssem0, ssem1, isem):
        c = lax.axis_index("c")
        s = lax.axis_index("s")
        wid = c * NS + s
        my_rows = pl.ds(s * rows_per_tile, rows_per_tile)

        pltpu.sync_copy(
            src_hbm.at[pl.ds(wid * chunks_per_tile, chunks_per_tile)], src_v
        )
        pltpu.sync_copy(
            dst_hbm.at[pl.ds(wid * chunks_per_tile, chunks_per_tile)], dst_v
        )

        def zrow(i, carry):
            for jj in range(d // 16):
                zb_v[i, pl.ds(jj * 16, 16)] = jnp.zeros((16,), jnp.float32)
            return carry

        lax.fori_loop(0, ZB, zrow, 0)
        for r in range(reps):
            pltpu.sync_copy(zb_v, acc_sh.at[pl.ds(s * rows_per_tile + r * ZB, ZB)])
        plsc.subcore_barrier()

        def fire_gathers(half, grp, sem):
            return [
                pltpu.async_copy(
                    xb_hbm.at[src_v.at[grp * GRP + b]], rows_v.at[half, b],
                    sem)
                for b in range(GRP)
            ]

        def fire_scatters(half, grp, sem):
            return [
                pltpu.async_copy(
                    rows_v.at[half, b], acc_sh.at[dst_v.at[grp * GRP + b]],
                    sem, add=True)
                for b in range(GRP)
            ]

        def body(i, carry):
            ga = 2 * i
            gb = 2 * i + 1
            gA = fire_gathers(0, ga, gsem0)
            for cp in gA:
                cp.wait()
            sA = fire_scatters(0, ga, ssem0)
            gB = fire_gathers(1, gb, gsem1)
            for cp in gB:
                cp.wait()
            for cp in sA:
                cp.wait()
            sB = fire_scatters(1, gb, ssem1)
            for cp in sB:
                cp.wait()
            return carry

        lax.fori_loop(0, niter, body, 0)
        plsc.subcore_barrier()

        pltpu.sync_copy(acc_sh.at[my_rows], out_hbm.at[c].at[my_rows])

    return agg


def _gin_mlp(n, sfull, w1_ref, b1_ref, g_ref, be_ref, w2_ref, b2_ref,
             out_ref):
    t = jnp.dot(sfull, w1_ref[...], preferred_element_type=jnp.float32)
    t = t + b1_ref[...]
    mu = jnp.sum(t, axis=0, keepdims=True) * (1.0 / n)
    ctr = t - mu
    var = jnp.sum(ctr * ctr, axis=0, keepdims=True) * (1.0 / n)
    h = ctr * lax.rsqrt(var + 1e-5) * g_ref[...] + be_ref[...]
    h = jnp.maximum(h, 0.0)
    h = jnp.dot(h, w2_ref[...], preferred_element_type=jnp.float32) + b2_ref[...]
    out_ref[...] = jnp.maximum(h, 0.0)


def _gin_dense1_body(n, x_ref, pa_ref, pb_ref, w1_ref, b1_ref, g_ref,
                     be_ref, w2_ref, b2_ref, out_ref):
    agg = jnp.concatenate(
        [pa_ref[0][:n] + pa_ref[1][:n], pb_ref[0][:n] + pb_ref[1][:n]], axis=1
    )
    sfull = x_ref[...] + agg
    _gin_mlp(n, sfull, w1_ref, b1_ref, g_ref, be_ref, w2_ref, b2_ref, out_ref)


def _gin_dense_body(n, x_ref, a_ref, w1_ref, b1_ref, g_ref, be_ref,
                    w2_ref, b2_ref, out_ref):
    sfull = x_ref[...] + a_ref[0][:n] + a_ref[1][:n]
    _gin_mlp(n, sfull, w1_ref, b1_ref, g_ref, be_ref, w2_ref, b2_ref, out_ref)


@functools.cache
def _make_gin_dense1(n, h):
    return pl.pallas_call(
        functools.partial(_gin_dense1_body, n),
        out_shape=jax.ShapeDtypeStruct((n, h), jnp.float32),
    )


@functools.cache
def _make_gin_dense(n, h):
    return pl.pallas_call(
        functools.partial(_gin_dense_body, n),
        out_shape=jax.ShapeDtypeStruct((n, h), jnp.float32),
    )


def _dense3_pool_head_body(n, g, x_ref, a_ref, w1_ref, b1_ref, g_ref, be_ref,
                           w2_ref, b2_ref, h1_ref, h2_ref, batch_ref,
                           fc1w_ref, fc1b_ref, fc2w_ref, fc2b_ref, out_ref):
    sfull = x_ref[...] + a_ref[0][:n] + a_ref[1][:n]
    t = jnp.dot(sfull, w1_ref[...], preferred_element_type=jnp.float32)
    t = t + b1_ref[...]
    mu = jnp.sum(t, axis=0, keepdims=True) * (1.0 / n)
    ctr = t - mu
    var = jnp.sum(ctr * ctr, axis=0, keepdims=True) * (1.0 / n)
    hh = ctr * lax.rsqrt(var + 1e-5) * g_ref[...] + be_ref[...]
    hh = jnp.maximum(hh, 0.0)
    hh = jnp.dot(hh, w2_ref[...], preferred_element_type=jnp.float32)
    h3 = jnp.maximum(hh + b2_ref[...], 0.0)

    b = batch_ref[...]
    gids = lax.broadcasted_iota(jnp.int32, (g, n), 0)
    sel = jnp.where(gids == b, 1.0, 0.0)
    p1 = jnp.dot(sel, h1_ref[...], preferred_element_type=jnp.float32)
    p2 = jnp.dot(sel, h2_ref[...], preferred_element_type=jnp.float32)
    p3 = jnp.dot(sel, h3, preferred_element_type=jnp.float32)
    cat = jnp.concatenate([p1, p2, p3], axis=1)
    y = jnp.dot(cat, fc1w_ref[...], preferred_element_type=jnp.float32)
    y = jnp.maximum(y + fc1b_ref[...], 0.0)
    y = jnp.dot(y, fc2w_ref[...], preferred_element_type=jnp.float32)
    y = y + fc2b_ref[...]
    m = jnp.max(y, axis=1, keepdims=True)
    ex = jnp.exp(y - m)
    out_ref[...] = (y - m) - jnp.log(jnp.sum(ex, axis=1, keepdims=True))


@functools.cache
def _make_dense3_pool_head(n, g, out):
    return pl.pallas_call(
        functools.partial(_dense3_pool_head_body, n, g),
        out_shape=jax.ShapeDtypeStruct((g, out), jnp.float32),
    )


def kernel(x, edge_index, batch, c1_W1, c1_b1, c1_g, c1_be, c1_W2, c1_b2,
           c2_W1, c2_b1, c2_g, c2_be, c2_W2, c2_b2, c3_W1, c3_b1, c3_g,
           c3_be, c3_W2, c3_b2, fc1_W, fc1_b, fc2_W, fc2_b):
    n, d = x.shape
    e = edge_index.shape[1]
    h = c1_W1.shape[1]
    g = 64
    out = fc2_W.shape[1]
    _, n_pad = _pad_rows(n)

    src2d = edge_index[0].reshape(e // CH, CH)
    dst2d = edge_index[1].reshape(e // CH, CH)

    xa = x[:, :h]
    xb = x[:, h:]

    agg = _make_agg(n, n_pad, h, e)
    dense1 = _make_gin_dense1(n, h)
    dense23 = _make_gin_dense(n, h)
    dense3_pool = _make_dense3_pool_head(n, g, out)

    pa = agg(xa, src2d, dst2d)
    pb = agg(xb, src2d, dst2d)
    h1 = dense1(x, pa, pb, c1_W1, c1_b1.reshape(1, h), c1_g.reshape(1, h),
                c1_be.reshape(1, h), c1_W2, c1_b2.reshape(1, h))

    parts2 = agg(h1, src2d, dst2d)
    h2 = dense23(h1, parts2, c2_W1, c2_b1.reshape(1, h), c2_g.reshape(1, h),
                 c2_be.reshape(1, h), c2_W2, c2_b2.reshape(1, h))

    parts3 = agg(h2, src2d, dst2d)
    return dense3_pool(h2, parts3, c3_W1, c3_b1.reshape(1, h),
                       c3_g.reshape(1, h), c3_be.reshape(1, h), c3_W2,
                       c3_b2.reshape(1, h), h1, h2, batch.reshape(1, n),
                       fc1_W, fc1_b.reshape(1, 3 * h), fc2_W,
                       fc2_b.reshape(1, out))
